# baseline (speedup 1.0000x reference)
import jax
import jax.numpy as jnp
from jax import lax
from jax.experimental import pallas as pl
from jax.experimental.pallas import tpu as pltpu

K = 16
TILE = 256


def _dot(a, b, dnums):
    return lax.dot_general(a, b, dnums, preferred_element_type=jnp.float32)


def _knn_edge_stage(f, W, b):
    C, N = f.shape
    ch = W.shape[0]
    Wa = W[:, :C]
    Wb = W[:, C:]
    cn = (((1,), (0,)), ((), ()))
    v = _dot(Wb, f, cn)
    u = _dot(Wa - Wb, f, cn) + b
    sq = jnp.sum(f * f, axis=0, keepdims=True)

    iota = lax.broadcasted_iota(jnp.int32, (TILE, N), 1)

    def extract(_, carry):
        d, macc = carry
        mn = jnp.min(d, axis=1, keepdims=True)
        tied = d <= mn
        jmin = jnp.min(jnp.where(tied, iota, N), axis=1, keepdims=True)
        onehot = (iota == jmin)
        vsel = _dot(v, onehot.astype(jnp.float32),
                    (((1,), (1,)), ((), ())))
        macc = jnp.maximum(macc, vsel)
        d = jnp.where(onehot, jnp.inf, d)
        return d, macc

    tiles = []
    for t in range(N // TILE):
        fr = f[:, t * TILE:(t + 1) * TILE]
        inner = _dot(fr, f, (((0,), (0,)), ((), ())))
        d = sq - 2.0 * inner
        _, macc = lax.fori_loop(
            0, K, extract,
            (d, jnp.full((ch, TILE), -jnp.inf, jnp.float32)))
        tiles.append(macc)
    m = jnp.concatenate(tiles, axis=1)
    return jax.nn.relu(u + m)


def _fwd(x_ref, W0_ref, b0_ref, W1_ref, b1_ref, W2_ref, b2_ref,
         Wf_ref, bf_ref, out_ref):
    x = x_ref[0]
    N = x.shape[1]
    f1 = _knn_edge_stage(x, W0_ref[...], b0_ref[...])
    f2 = _knn_edge_stage(f1, W1_ref[...], b1_ref[...]) + f1
    f3 = _knn_edge_stage(f2, W2_ref[...], b2_ref[...]) + f2

    Wf = Wf_ref[...]
    ch = f1.shape[0]
    cn = (((1,), (0,)), ((), ()))
    ff = (_dot(Wf[:, :ch], f1, cn) + _dot(Wf[:, ch:2 * ch], f2, cn)
          + _dot(Wf[:, 2 * ch:], f3, cn) + bf_ref[...])
    ff = jax.nn.relu(ff)
    fmax = jnp.max(ff, axis=1, keepdims=True)

    out_ref[0, 0:1024, :] = jnp.broadcast_to(fmax, (1024, N))
    out_ref[0, 1024:1088, :] = f1
    out_ref[0, 1088:1152, :] = f2
    out_ref[0, 1152:1216, :] = f3


def kernel(inputs, W0, b0, W1, b1, W2, b2, Wf, bf):
    x = inputs[..., 0]
    B, Cin, N = x.shape
    ch = W0.shape[0]
    b0c = b0.reshape(ch, 1)
    b1c = b1.reshape(ch, 1)
    b2c = b2.reshape(ch, 1)
    bfc = bf.reshape(-1, 1)
    Cout = Wf.shape[0] + 3 * ch

    full = lambda a: pl.BlockSpec(a.shape, lambda b: (0,) * a.ndim)
    out = pl.pallas_call(
        _fwd,
        grid=(B,),
        in_specs=[
            pl.BlockSpec((1, Cin, N), lambda b: (b, 0, 0)),
            full(W0), full(b0c), full(W1), full(b1c),
            full(W2), full(b2c), full(Wf), full(bfc),
        ],
        out_specs=pl.BlockSpec((1, Cout, N), lambda b: (b, 0, 0)),
        out_shape=jax.ShapeDtypeStruct((B, Cout, N), jnp.float32),
        compiler_params=pltpu.CompilerParams(
            dimension_semantics=("parallel",)),
    )(x, W0, b0c, W1, b1c, W2, b2c, Wf, bfc)
    return out[..., None]


# speedup vs baseline: 1.0699x; 1.0699x over previous
import jax
import jax.numpy as jnp
from jax import lax
from jax.experimental import pallas as pl
from jax.experimental.pallas import tpu as pltpu

K = 16
TILE = 512


def _dot(a, b, dnums):
    return lax.dot_general(a, b, dnums, preferred_element_type=jnp.float32)


def _knn_edge_stage(f, W, b):
    C, N = f.shape
    ch = W.shape[0]
    Wa = W[:, :C]
    Wb = W[:, C:]
    cn = (((1,), (0,)), ((), ()))
    v = _dot(Wb, f, cn)
    u = _dot(Wa - Wb, f, cn) + b
    sq = jnp.sum(f * f, axis=0, keepdims=True)

    iota = lax.broadcasted_iota(jnp.int32, (TILE, N), 1)

    def extract(_, carry):
        d, macc = carry
        mn = jnp.min(d, axis=1, keepdims=True)
        tied = d <= mn
        jmin = jnp.min(jnp.where(tied, iota, N), axis=1, keepdims=True)
        onehot = (iota == jmin)
        vsel = _dot(v, onehot.astype(jnp.float32),
                    (((1,), (1,)), ((), ())))
        macc = jnp.maximum(macc, vsel)
        d = jnp.where(onehot, jnp.inf, d)
        return d, macc

    tiles = []
    for t in range(N // TILE):
        fr = f[:, t * TILE:(t + 1) * TILE]
        inner = _dot(fr, f, (((0,), (0,)), ((), ())))
        d = sq - 2.0 * inner
        _, macc = lax.fori_loop(
            0, K, extract,
            (d, jnp.full((ch, TILE), -jnp.inf, jnp.float32)))
        tiles.append(macc)
    m = jnp.concatenate(tiles, axis=1)
    return jax.nn.relu(u + m)


def _fwd(x_ref, W0_ref, b0_ref, W1_ref, b1_ref, W2_ref, b2_ref,
         Wf_ref, bf_ref, out_ref):
    x = x_ref[0]
    N = x.shape[1]
    f1 = _knn_edge_stage(x, W0_ref[...], b0_ref[...])
    f2 = _knn_edge_stage(f1, W1_ref[...], b1_ref[...]) + f1
    f3 = _knn_edge_stage(f2, W2_ref[...], b2_ref[...]) + f2

    Wf = Wf_ref[...]
    ch = f1.shape[0]
    cn = (((1,), (0,)), ((), ()))
    ff = (_dot(Wf[:, :ch], f1, cn) + _dot(Wf[:, ch:2 * ch], f2, cn)
          + _dot(Wf[:, 2 * ch:], f3, cn) + bf_ref[...])
    ff = jax.nn.relu(ff)
    fmax = jnp.max(ff, axis=1, keepdims=True)

    out_ref[0, 0:1024, :] = jnp.broadcast_to(fmax, (1024, N))
    out_ref[0, 1024:1088, :] = f1
    out_ref[0, 1088:1152, :] = f2
    out_ref[0, 1152:1216, :] = f3


def kernel(inputs, W0, b0, W1, b1, W2, b2, Wf, bf):
    x = inputs[..., 0]
    B, Cin, N = x.shape
    ch = W0.shape[0]
    b0c = b0.reshape(ch, 1)
    b1c = b1.reshape(ch, 1)
    b2c = b2.reshape(ch, 1)
    bfc = bf.reshape(-1, 1)
    Cout = Wf.shape[0] + 3 * ch

    full = lambda a: pl.BlockSpec(a.shape, lambda b: (0,) * a.ndim)
    out = pl.pallas_call(
        _fwd,
        grid=(B,),
        in_specs=[
            pl.BlockSpec((1, Cin, N), lambda b: (b, 0, 0)),
            full(W0), full(b0c), full(W1), full(b1c),
            full(W2), full(b2c), full(Wf), full(bfc),
        ],
        out_specs=pl.BlockSpec((1, Cout, N), lambda b: (b, 0, 0)),
        out_shape=jax.ShapeDtypeStruct((B, Cout, N), jnp.float32),
        compiler_params=pltpu.CompilerParams(
            dimension_semantics=("parallel",)),
    )(x, W0, b0c, W1, b1c, W2, b2c, Wf, bfc)
    return out[..., None]


# speedup vs baseline: 1.1542x; 1.0788x over previous
import jax
import jax.numpy as jnp
from jax import lax
from jax.experimental import pallas as pl
from jax.experimental.pallas import tpu as pltpu

K = 16
TILE = 1024


def _dot(a, b, dnums):
    return lax.dot_general(a, b, dnums, preferred_element_type=jnp.float32)


def _knn_edge_stage(f, W, b):
    C, N = f.shape
    ch = W.shape[0]
    Wa = W[:, :C]
    Wb = W[:, C:]
    cn = (((1,), (0,)), ((), ()))
    v = _dot(Wb, f, cn)
    u = _dot(Wa - Wb, f, cn) + b
    sq = jnp.sum(f * f, axis=0, keepdims=True)

    iota = lax.broadcasted_iota(jnp.int32, (TILE, N), 1)

    def extract(_, carry):
        d, macc = carry
        mn = jnp.min(d, axis=1, keepdims=True)
        tied = d <= mn
        jmin = jnp.min(jnp.where(tied, iota, N), axis=1, keepdims=True)
        onehot = (iota == jmin)
        vsel = _dot(v, onehot.astype(jnp.float32),
                    (((1,), (1,)), ((), ())))
        macc = jnp.maximum(macc, vsel)
        d = jnp.where(onehot, jnp.inf, d)
        return d, macc

    tiles = []
    for t in range(N // TILE):
        fr = f[:, t * TILE:(t + 1) * TILE]
        inner = _dot(fr, f, (((0,), (0,)), ((), ())))
        d = sq - 2.0 * inner
        _, macc = lax.fori_loop(
            0, K, extract,
            (d, jnp.full((ch, TILE), -jnp.inf, jnp.float32)))
        tiles.append(macc)
    m = jnp.concatenate(tiles, axis=1)
    return jax.nn.relu(u + m)


def _fwd(x_ref, W0_ref, b0_ref, W1_ref, b1_ref, W2_ref, b2_ref,
         Wf_ref, bf_ref, out_ref):
    x = x_ref[0]
    N = x.shape[1]
    f1 = _knn_edge_stage(x, W0_ref[...], b0_ref[...])
    f2 = _knn_edge_stage(f1, W1_ref[...], b1_ref[...]) + f1
    f3 = _knn_edge_stage(f2, W2_ref[...], b2_ref[...]) + f2

    Wf = Wf_ref[...]
    ch = f1.shape[0]
    cn = (((1,), (0,)), ((), ()))
    ff = (_dot(Wf[:, :ch], f1, cn) + _dot(Wf[:, ch:2 * ch], f2, cn)
          + _dot(Wf[:, 2 * ch:], f3, cn) + bf_ref[...])
    ff = jax.nn.relu(ff)
    fmax = jnp.max(ff, axis=1, keepdims=True)

    out_ref[0, 0:1024, :] = jnp.broadcast_to(fmax, (1024, N))
    out_ref[0, 1024:1088, :] = f1
    out_ref[0, 1088:1152, :] = f2
    out_ref[0, 1152:1216, :] = f3


def kernel(inputs, W0, b0, W1, b1, W2, b2, Wf, bf):
    x = inputs[..., 0]
    B, Cin, N = x.shape
    ch = W0.shape[0]
    b0c = b0.reshape(ch, 1)
    b1c = b1.reshape(ch, 1)
    b2c = b2.reshape(ch, 1)
    bfc = bf.reshape(-1, 1)
    Cout = Wf.shape[0] + 3 * ch

    full = lambda a: pl.BlockSpec(a.shape, lambda b: (0,) * a.ndim)
    out = pl.pallas_call(
        _fwd,
        grid=(B,),
        in_specs=[
            pl.BlockSpec((1, Cin, N), lambda b: (b, 0, 0)),
            full(W0), full(b0c), full(W1), full(b1c),
            full(W2), full(b2c), full(Wf), full(bfc),
        ],
        out_specs=pl.BlockSpec((1, Cout, N), lambda b: (b, 0, 0)),
        out_shape=jax.ShapeDtypeStruct((B, Cout, N), jnp.float32),
        compiler_params=pltpu.CompilerParams(
            dimension_semantics=("parallel",)),
    )(x, W0, b0c, W1, b1c, W2, b2c, Wf, bfc)
    return out[..., None]


# speedup vs baseline: 1.2808x; 1.1096x over previous
import jax
import jax.numpy as jnp
from jax import lax
from jax.experimental import pallas as pl
from jax.experimental.pallas import tpu as pltpu

K = 16
TILE = 1024


def _dot(a, b, dnums):
    return lax.dot_general(a, b, dnums, preferred_element_type=jnp.float32)


def _knn_edge_stage(f, W, b):
    C, N = f.shape
    ch = W.shape[0]
    Wa = W[:, :C]
    Wb = W[:, C:]
    cn = (((1,), (0,)), ((), ()))
    v = _dot(Wb, f, cn)
    u = _dot(Wa - Wb, f, cn) + b
    sq = jnp.sum(f * f, axis=0, keepdims=True)

    iota = lax.broadcasted_iota(jnp.int32, (TILE, N), 1)

    def extract(_, carry):
        d, macc = carry
        jmin = jnp.argmin(d, axis=1).reshape(TILE, 1)
        onehot = (iota == jmin)
        vsel = _dot(v, jnp.where(onehot, 1.0, 0.0),
                    (((1,), (1,)), ((), ())))
        macc = jnp.maximum(macc, vsel)
        d = jnp.where(onehot, jnp.inf, d)
        return d, macc

    tiles = []
    for t in range(N // TILE):
        fr = f[:, t * TILE:(t + 1) * TILE]
        inner = _dot(fr, f, (((0,), (0,)), ((), ())))
        d = sq - 2.0 * inner
        _, macc = lax.fori_loop(
            0, K, extract,
            (d, jnp.full((ch, TILE), -jnp.inf, jnp.float32)))
        tiles.append(macc)
    m = jnp.concatenate(tiles, axis=1)
    return jax.nn.relu(u + m)


def _fwd(x_ref, W0_ref, b0_ref, W1_ref, b1_ref, W2_ref, b2_ref,
         Wf_ref, bf_ref, out_ref):
    x = x_ref[0]
    N = x.shape[1]
    f1 = _knn_edge_stage(x, W0_ref[...], b0_ref[...])
    f2 = _knn_edge_stage(f1, W1_ref[...], b1_ref[...]) + f1
    f3 = _knn_edge_stage(f2, W2_ref[...], b2_ref[...]) + f2

    Wf = Wf_ref[...]
    ch = f1.shape[0]
    cn = (((1,), (0,)), ((), ()))
    ff = (_dot(Wf[:, :ch], f1, cn) + _dot(Wf[:, ch:2 * ch], f2, cn)
          + _dot(Wf[:, 2 * ch:], f3, cn) + bf_ref[...])
    ff = jax.nn.relu(ff)
    fmax = jnp.max(ff, axis=1, keepdims=True)

    out_ref[0, 0:1024, :] = jnp.broadcast_to(fmax, (1024, N))
    out_ref[0, 1024:1088, :] = f1
    out_ref[0, 1088:1152, :] = f2
    out_ref[0, 1152:1216, :] = f3


def kernel(inputs, W0, b0, W1, b1, W2, b2, Wf, bf):
    x = inputs[..., 0]
    B, Cin, N = x.shape
    ch = W0.shape[0]
    b0c = b0.reshape(ch, 1)
    b1c = b1.reshape(ch, 1)
    b2c = b2.reshape(ch, 1)
    bfc = bf.reshape(-1, 1)
    Cout = Wf.shape[0] + 3 * ch

    full = lambda a: pl.BlockSpec(a.shape, lambda b: (0,) * a.ndim)
    out = pl.pallas_call(
        _fwd,
        grid=(B,),
        in_specs=[
            pl.BlockSpec((1, Cin, N), lambda b: (b, 0, 0)),
            full(W0), full(b0c), full(W1), full(b1c),
            full(W2), full(b2c), full(Wf), full(bfc),
        ],
        out_specs=pl.BlockSpec((1, Cout, N), lambda b: (b, 0, 0)),
        out_shape=jax.ShapeDtypeStruct((B, Cout, N), jnp.float32),
        compiler_params=pltpu.CompilerParams(
            dimension_semantics=("parallel",)),
    )(x, W0, b0c, W1, b1c, W2, b2c, Wf, bfc)
    return out[..., None]


# speedup vs baseline: 1.9473x; 1.5204x over previous
import functools

import jax
import jax.numpy as jnp
from jax import lax
from jax.experimental import pallas as pl
from jax.experimental.pallas import tpu as pltpu
from jax.experimental.pallas import tpu_sc as plsc

K = 16
TILE = 1024
CH = 64


def _dot(a, b, dnums):
    return lax.dot_general(a, b, dnums, preferred_element_type=jnp.float32)


_CN = (((1,), (0,)), ((), ()))


def _knn_indices(f, idx_ref, boff):
    C, N = f.shape
    sq = jnp.sum(f * f, axis=0, keepdims=True)
    iota = lax.broadcasted_iota(jnp.int32, (TILE, N), 1)
    for t in range(N // TILE):
        fr = f[:, t * TILE:(t + 1) * TILE]
        inner = _dot(fr, f, (((0,), (0,)), ((), ())))
        d = sq - 2.0 * inner
        rounds = []
        for r in range(K):
            jmin = jnp.argmin(d, axis=1)
            rounds.append((jmin + boff).reshape(1, TILE))
            d = jnp.where(iota == jmin.reshape(TILE, 1), jnp.inf, d)
        idx_ref[0, :, t * TILE:(t + 1) * TILE] = jnp.concatenate(rounds, 0)


def _uv(f, W, b):
    C = f.shape[0]
    Wa = W[:, :C]
    Wb = W[:, C:]
    v = _dot(Wb, f, _CN)
    u = _dot(Wa - Wb, f, _CN) + b
    return u, v


def _store_v_padded(v_ref, v):
    v_ref[0, 0:CH, :] = v
    v_ref[0, CH:2 * CH, :] = jnp.zeros_like(v)


def _tc1(x_ref, W0_ref, b0_ref, u_ref, v_ref, idx_ref):
    x = x_ref[0]
    N = x.shape[1]
    u, v = _uv(x, W0_ref[...], b0_ref[...])
    u_ref[0] = u
    _store_v_padded(v_ref, v)
    _knn_indices(x, idx_ref, pl.program_id(0) * N)


def _tc_mid(up_ref, mp_ref, fp_ref, W_ref, b_ref,
            f_ref, u_ref, v_ref, idx_ref, *, residual):
    m = jnp.transpose(mp_ref[0], (1, 0))
    f = jax.nn.relu(up_ref[0] + m)
    if residual:
        f = f + fp_ref[0]
    N = f.shape[1]
    f_ref[0] = f
    u, v = _uv(f, W_ref[...], b_ref[...])
    u_ref[0] = u
    _store_v_padded(v_ref, v)
    _knn_indices(f, idx_ref, pl.program_id(0) * N)


def _tc_final(u2_ref, m2_ref, f2_ref, f1_ref, Wf_ref, bf_ref, out_ref):
    m2 = jnp.transpose(m2_ref[0], (1, 0))
    f2 = f2_ref[0]
    f1 = f1_ref[0]
    f3 = jax.nn.relu(u2_ref[0] + m2) + f2
    N = f3.shape[1]
    Wf = Wf_ref[...]
    ff = (_dot(Wf[:, :CH], f1, _CN) + _dot(Wf[:, CH:2 * CH], f2, _CN)
          + _dot(Wf[:, 2 * CH:], f3, _CN) + bf_ref[...])
    ff = jax.nn.relu(ff)
    fmax = jnp.max(ff, axis=1, keepdims=True)
    out_ref[0, 0:1024, :] = jnp.broadcast_to(fmax, (1024, N))
    out_ref[0, 1024:1088, :] = f1
    out_ref[0, 1088:1152, :] = f2
    out_ref[0, 1152:1216, :] = f3


def _make_sc_gather_max(npts):
    info = plsc.get_sparse_core_info()
    NC, NS, L = info.num_cores, info.num_subcores, info.num_lanes
    NW = NC * NS
    PW = npts // NW
    CHP = 2 * CH
    CPG = 128 // K
    GPC = 8
    HG = GPC // 2
    CP = CPG * GPC
    NCHUNK = PW // CP

    @functools.partial(
        pl.kernel,
        mesh=plsc.VectorSubcoreMesh(core_axis_name="c", subcore_axis_name="s"),
        out_type=jax.ShapeDtypeStruct((npts, CH), jnp.float32),
        scratch_types=[
            pltpu.VMEM((GPC, CPG * K), jnp.int32),
            pltpu.VMEM((HG * CPG * K, CHP), jnp.float32),
            pltpu.VMEM((CP, CH), jnp.float32),
            pltpu.SemaphoreType.DMA,
        ],
    )
    def sc_gather_max(tab_hbm, ix_hbm, out_hbm, idx_v, rows_v, m_v, sem):
        wid = lax.axis_index("s") * NC + lax.axis_index("c")
        base = wid * PW

        def chunk_body(g, carry):
            p0 = pl.multiple_of(base + g * CP, CP)
            pltpu.sync_copy(
                ix_hbm.at[pl.ds(pl.multiple_of(p0 // CPG, GPC), GPC)], idx_v)
            for h in range(2):
                copies = [
                    pltpu.async_copy(
                        tab_hbm.at[idx_v.at[h * HG + j]],
                        rows_v.at[pl.ds(j * CPG * K, CPG * K)], sem)
                    for j in range(HG)
                ]
                for c in copies:
                    c.wait()
                pbase = h * (CP // 2)

                def pbody(p, carry2):
                    for c in range(CH // L):
                        a = rows_v[p * K, pl.ds(c * L, L)]
                        for r in range(1, K):
                            a = jnp.maximum(
                                a, rows_v[p * K + r, pl.ds(c * L, L)])
                        m_v[pbase + p, pl.ds(c * L, L)] = a
                    return carry2

                lax.fori_loop(0, CP // 2, pbody, 0)
            pltpu.sync_copy(m_v, out_hbm.at[pl.ds(pl.multiple_of(p0, CP), CP)])
            return carry

        lax.fori_loop(0, NCHUNK, chunk_body, 0)

    return sc_gather_max


def kernel(inputs, W0, b0, W1, b1, W2, b2, Wf, bf):
    x = inputs[..., 0]
    B, Cin, N = x.shape
    npts = B * N
    b0c = b0.reshape(CH, 1)
    b1c = b1.reshape(CH, 1)
    b2c = b2.reshape(CH, 1)
    bfc = bf.reshape(-1, 1)
    Cout = Wf.shape[0] + 3 * CH

    full = lambda a: pl.BlockSpec(a.shape, lambda b: (0,) * a.ndim)
    bblk = lambda c, dt=jnp.float32: (
        pl.BlockSpec((1, c, N), lambda b: (b, 0, 0)),
        jax.ShapeDtypeStruct((B, c, N), dt))
    params = pltpu.CompilerParams(dimension_semantics=("parallel",))
    sc_gather_max = _make_sc_gather_max(npts)

    def to_sc(v, idx):
        tab = v.transpose(0, 2, 1).reshape(npts, 2 * CH)
        ix2 = idx.transpose(0, 2, 1).reshape(npts // (128 // K), 128)
        m = sc_gather_max(tab, ix2)
        return m.reshape(B, N, CH)

    ub, us = bblk(CH)
    vb, vs = bblk(2 * CH)
    ib, is_ = bblk(K, jnp.int32)
    mb = pl.BlockSpec((1, N, CH), lambda b: (b, 0, 0))

    u0, v0, idx0 = pl.pallas_call(
        _tc1, grid=(B,),
        in_specs=[pl.BlockSpec((1, Cin, N), lambda b: (b, 0, 0)),
                  full(W0), full(b0c)],
        out_specs=[ub, vb, ib], out_shape=[us, vs, is_],
        compiler_params=params,
    )(x, W0, b0c)
    m0 = to_sc(v0, idx0)

    f1, u1, v1, idx1 = pl.pallas_call(
        functools.partial(_tc_mid, residual=False), grid=(B,),
        in_specs=[ub, mb, ub, full(W1), full(b1c)],
        out_specs=[ub, ub, vb, ib], out_shape=[us, us, vs, is_],
        compiler_params=params,
    )(u0, m0, u0, W1, b1c)
    m1 = to_sc(v1, idx1)

    f2, u2, v2, idx2 = pl.pallas_call(
        functools.partial(_tc_mid, residual=True), grid=(B,),
        in_specs=[ub, mb, ub, full(W2), full(b2c)],
        out_specs=[ub, ub, vb, ib], out_shape=[us, us, vs, is_],
        compiler_params=params,
    )(u1, m1, f1, W2, b2c)
    m2 = to_sc(v2, idx2)

    out = pl.pallas_call(
        _tc_final, grid=(B,),
        in_specs=[ub, mb, ub, ub, full(Wf), full(bfc)],
        out_specs=pl.BlockSpec((1, Cout, N), lambda b: (b, 0, 0)),
        out_shape=jax.ShapeDtypeStruct((B, Cout, N), jnp.float32),
        compiler_params=params,
    )(u2, m2, f2, f1, Wf, bfc)
    return out[..., None]
